# baseline (device time: 14644 ns/iter reference)
import jax
import jax.numpy as jnp
from jax import lax
from jax.experimental import pallas as pl
from jax.experimental.pallas import tpu as pltpu

N_DEV = 16


def kernel(x, dy, gamma):
    m, d = x.shape

    def body(x_hbm, dy_hbm, gamma_hbm, out_hbm, xv_ref, dyv_ref, out_v,
             send_buf, recv_buf, local_sems, send_sems, recv_sems):
        my_pos = lax.axis_index("i")

        h = m // 2
        cps = []
        for i, (src, dst) in enumerate(((x_hbm, xv_ref), (dy_hbm, dyv_ref))):
            for j in range(2):
                cp = pltpu.make_async_copy(
                    src.at[pl.ds(j * h, h), :],
                    dst.at[pl.ds(j * h, h), :],
                    local_sems.at[2 * i + j],
                )
                cp.start()
                cps.append(cp)

        barrier_sem = pltpu.get_barrier_semaphore()
        for dd in range(1, N_DEV):
            peer = lax.rem(my_pos + dd, N_DEV)
            pl.semaphore_signal(
                barrier_sem, inc=1,
                device_id=(peer,), device_id_type=pl.DeviceIdType.MESH,
            )

        for cp in cps:
            cp.wait()

        xv = xv_ref[:, :]
        dyv = dyv_ref[:, :]
        mu = jnp.mean(xv, axis=1, keepdims=True)
        var = jnp.mean(xv * xv, axis=1, keepdims=True) - mu * mu
        rstd = lax.rsqrt(var + 1e-5)
        xhat = (xv - mu) * rstd
        dgamma = jnp.sum(dyv * xhat, axis=0, keepdims=True)
        dbeta = jnp.sum(dyv, axis=0, keepdims=True)
        partial = jnp.concatenate([dgamma, dbeta], axis=0)
        send_buf[:, :] = partial

        pl.semaphore_wait(barrier_sem, N_DEV - 1)

        rdmas = []
        for dd in range(1, N_DEV):
            peer = lax.rem(my_pos + dd, N_DEV)
            rdma = pltpu.make_async_remote_copy(
                src_ref=send_buf,
                dst_ref=recv_buf.at[dd - 1],
                send_sem=send_sems.at[dd - 1],
                recv_sem=recv_sems.at[dd - 1],
                device_id=(peer,),
                device_id_type=pl.DeviceIdType.MESH,
            )
            rdma.start()
            rdmas.append(rdma)

        for rdma in rdmas:
            rdma.wait_recv()
        out_v[:, :] = partial + jnp.sum(recv_buf[:, :, :], axis=0)

        cp_out = pltpu.make_async_copy(out_v, out_hbm, local_sems.at[4])
        cp_out.start()
        cp_out.wait()

        for rdma in rdmas:
            rdma.wait_send()

    return pl.pallas_call(
        body,
        out_shape=jax.ShapeDtypeStruct((2, d), jnp.float32),
        in_specs=[
            pl.BlockSpec(memory_space=pl.ANY),
            pl.BlockSpec(memory_space=pl.ANY),
            pl.BlockSpec(memory_space=pl.ANY),
        ],
        out_specs=pl.BlockSpec(memory_space=pl.ANY),
        scratch_shapes=[
            pltpu.VMEM((m, d), jnp.float32),
            pltpu.VMEM((m, d), jnp.float32),
            pltpu.VMEM((2, d), jnp.float32),
            pltpu.VMEM((2, d), jnp.float32),
            pltpu.VMEM((N_DEV - 1, 2, d), jnp.float32),
            pltpu.SemaphoreType.DMA((5,)),
            pltpu.SemaphoreType.DMA((N_DEV - 1,)),
            pltpu.SemaphoreType.DMA((N_DEV - 1,)),
        ],
        compiler_params=pltpu.CompilerParams(
            collective_id=0,
            vmem_limit_bytes=96 * 1024 * 1024,
        ),
    )(x, dy, gamma)


# device time: 14604 ns/iter; 1.0027x vs baseline; 1.0027x over previous
import jax
import jax.numpy as jnp
from jax import lax
from jax.experimental import pallas as pl
from jax.experimental.pallas import tpu as pltpu

N_DEV = 16


def kernel(x, dy, gamma):
    m, d = x.shape

    def body(x_hbm, dy_hbm, gamma_hbm, out_ref, xv_ref, dyv_ref,
             send_buf, recv_buf, local_sems, send_sems, recv_sems):
        my_pos = lax.axis_index("i")

        cps = [
            pltpu.make_async_copy(x_hbm, xv_ref, local_sems.at[0]),
            pltpu.make_async_copy(dy_hbm, dyv_ref, local_sems.at[1]),
        ]
        for cp in cps:
            cp.start()

        barrier_sem = pltpu.get_barrier_semaphore()
        for dd in range(1, N_DEV):
            peer = lax.rem(my_pos + dd, N_DEV)
            pl.semaphore_signal(
                barrier_sem, inc=1,
                device_id=(peer,), device_id_type=pl.DeviceIdType.MESH,
            )

        for cp in cps:
            cp.wait()

        xv = xv_ref[:, :]
        dyv = dyv_ref[:, :]
        mu = jnp.mean(xv, axis=1, keepdims=True)
        var = jnp.mean(xv * xv, axis=1, keepdims=True) - mu * mu
        rstd = lax.rsqrt(var + 1e-5)
        xhat = (xv - mu) * rstd
        dgamma = jnp.sum(dyv * xhat, axis=0, keepdims=True)
        dbeta = jnp.sum(dyv, axis=0, keepdims=True)
        partial = jnp.concatenate([dgamma, dbeta], axis=0)
        send_buf[:, :] = partial

        pl.semaphore_wait(barrier_sem, N_DEV - 1)

        rdmas = []
        for dd in range(1, N_DEV):
            peer = lax.rem(my_pos + dd, N_DEV)
            rdma = pltpu.make_async_remote_copy(
                src_ref=send_buf,
                dst_ref=recv_buf.at[dd - 1],
                send_sem=send_sems.at[dd - 1],
                recv_sem=recv_sems.at[dd - 1],
                device_id=(peer,),
                device_id_type=pl.DeviceIdType.MESH,
            )
            rdma.start()
            rdmas.append(rdma)

        for rdma in rdmas:
            rdma.wait_recv()
        out_ref[:, :] = partial + jnp.sum(recv_buf[:, :, :], axis=0)

        for rdma in rdmas:
            rdma.wait_send()

    return pl.pallas_call(
        body,
        out_shape=jax.ShapeDtypeStruct((2, d), jnp.float32),
        in_specs=[
            pl.BlockSpec(memory_space=pl.ANY),
            pl.BlockSpec(memory_space=pl.ANY),
            pl.BlockSpec(memory_space=pl.ANY),
        ],
        out_specs=pl.BlockSpec(memory_space=pltpu.VMEM),
        scratch_shapes=[
            pltpu.VMEM((m, d), jnp.float32),
            pltpu.VMEM((m, d), jnp.float32),
            pltpu.VMEM((2, d), jnp.float32),
            pltpu.VMEM((N_DEV - 1, 2, d), jnp.float32),
            pltpu.SemaphoreType.DMA((2,)),
            pltpu.SemaphoreType.DMA((N_DEV - 1,)),
            pltpu.SemaphoreType.DMA((N_DEV - 1,)),
        ],
        compiler_params=pltpu.CompilerParams(
            collective_id=0,
            vmem_limit_bytes=96 * 1024 * 1024,
        ),
    )(x, dy, gamma)
